# baseline (speedup 1.0000x reference)
import functools

import jax
import jax.numpy as jnp
from jax import lax
from jax.experimental import pallas as pl
from jax.experimental.pallas import tpu as pltpu
from jax.experimental.pallas import tpu_sc as plsc

V = 1000000
D = 64
B_POS = 16384
B_NEG = 81920
TOTAL = B_POS + B_NEG

NC, NS = 2, 16
NW = NC * NS
CHUNK = 512
SUB = 128
POS_PW = B_POS // NW
NEG_PW = B_NEG // NW
N_CHUNKS = (POS_PW + NEG_PW) // CHUNK

_mesh = plsc.VectorSubcoreMesh(
    core_axis_name="c", subcore_axis_name="s", num_cores=NC, num_subcores=NS)


@functools.partial(
    pl.kernel,
    out_type=jax.ShapeDtypeStruct((TOTAL,), jnp.float32),
    mesh=_mesh,
    compiler_params=pltpu.CompilerParams(
        needs_layout_passes=False, use_tc_tiling_on_sc=False),
    scratch_types=[
        pltpu.VMEM((CHUNK,), jnp.int32),
        pltpu.VMEM((CHUNK,), jnp.int32),
        pltpu.VMEM((CHUNK, D), jnp.float32),
        pltpu.VMEM((CHUNK, D), jnp.float32),
        pltpu.VMEM((CHUNK,), jnp.float32),
        pltpu.SemaphoreType.DMA,
    ],
)
def _dots_sc(pc, pt, ncx, ntg, ctx, tgt, out,
             idx_c, idx_t, rows_c, rows_t, dots_v, sem):
    wid = lax.axis_index("s") * NC + lax.axis_index("c")
    lanes16 = lax.iota(jnp.int32, 16)

    for chunk in range(N_CHUNKS):
        if chunk == 0:
            src_c, src_t = pc, pt
            base = wid * POS_PW
            out_base = base
        else:
            src_c, src_t = ncx, ntg
            base = wid * NEG_PW + (chunk - 1) * CHUNK
            out_base = B_POS + base

        pltpu.sync_copy(src_c.at[pl.ds(base, CHUNK)], idx_c)
        pltpu.sync_copy(src_t.at[pl.ds(base, CHUNK)], idx_t)
        handles = []
        for k in range(CHUNK // SUB):
            handles.append(pltpu.async_copy(
                ctx.at[idx_c.at[pl.ds(k * SUB, SUB)]],
                rows_c.at[pl.ds(k * SUB, SUB)], sem))
            handles.append(pltpu.async_copy(
                tgt.at[idx_t.at[pl.ds(k * SUB, SUB)]],
                rows_t.at[pl.ds(k * SUB, SUB)], sem))
        for h in handles:
            h.wait()

        def block_body(b, carry):
            tot = jnp.zeros((16,), jnp.float32)
            for i in range(16):
                p = b * 16 + i
                acc = rows_c[p, pl.ds(0, 16)] * rows_t[p, pl.ds(0, 16)]
                for q in range(1, 4):
                    acc = acc + (rows_c[p, pl.ds(q * 16, 16)]
                                 * rows_t[p, pl.ds(q * 16, 16)])
                s = jnp.sum(acc)
                tot = jnp.where(lanes16 == i, s, tot)
            dots_v[pl.ds(b * 16, 16)] = tot
            return carry

        lax.fori_loop(0, CHUNK // 16, block_body, 0)
        pltpu.sync_copy(dots_v, out.at[pl.ds(out_base, CHUNK)])


def _loss_tc(dp_ref, dn_ref, out_ref):
    dp = dp_ref[...]
    dn = dn_ref[...]
    pos_loss = -jnp.mean(jnp.log(jax.nn.sigmoid(dp)))
    neg_loss = -jnp.mean(jnp.log(jax.nn.sigmoid(-dn)))
    out_ref[0, 0] = pos_loss + neg_loss


_loss_call = pl.pallas_call(
    _loss_tc,
    out_shape=jax.ShapeDtypeStruct((1, 1), jnp.float32),
    out_specs=pl.BlockSpec(memory_space=pltpu.SMEM),
)


def kernel(positive_context, positive_target, negative_context,
           negative_target, context_embeddings, target_embeddings):
    pc = positive_context.astype(jnp.int32)
    pt = positive_target.astype(jnp.int32)
    ncx = negative_context.astype(jnp.int32)
    ntg = negative_target.astype(jnp.int32)
    dots = _dots_sc(pc, pt, ncx, ntg, context_embeddings, target_embeddings)
    dp = dots[:B_POS].reshape(B_POS // 128, 128)
    dn = dots[B_POS:].reshape(B_NEG // 128, 128)
    return _loss_call(dp, dn)[0, 0]


# speedup vs baseline: 1.1193x; 1.1193x over previous
import functools

import jax
import jax.numpy as jnp
from jax import lax
from jax.experimental import pallas as pl
from jax.experimental.pallas import tpu as pltpu
from jax.experimental.pallas import tpu_sc as plsc

V = 1000000
D = 64
B_POS = 16384
B_NEG = 81920
TOTAL = B_POS + B_NEG

NC, NS = 2, 16
NW = NC * NS
POS_PW = B_POS // NW
NEG_PW = B_NEG // NW
PAIRS_PW = POS_PW + NEG_PW
C = 16
NCH = PAIRS_PW // C
G = C // 16

_mesh = plsc.VectorSubcoreMesh(
    core_axis_name="c", subcore_axis_name="s", num_cores=NC, num_subcores=NS)


@functools.partial(
    pl.kernel,
    out_type=jax.ShapeDtypeStruct((TOTAL,), jnp.float32),
    mesh=_mesh,
    compiler_params=pltpu.CompilerParams(needs_layout_passes=False),
    scratch_types=[
        pltpu.VMEM((PAIRS_PW,), jnp.int32),
        pltpu.VMEM((PAIRS_PW,), jnp.int32),
        pltpu.VMEM((C * 8, D), jnp.float32),
        pltpu.VMEM((C * 8, D), jnp.float32),
        pltpu.VMEM((C * 8, D), jnp.float32),
        pltpu.VMEM((C * 8, D), jnp.float32),
        pltpu.VMEM((PAIRS_PW,), jnp.float32),
        pltpu.SemaphoreType.DMA,
        pltpu.SemaphoreType.DMA,
    ],
)
def _dots_sc(pc, pt, ncx, ntg, ctx, tgt, out,
             idx_c, idx_t, bc_a, bt_a, bc_b, bt_b, dots_v, sem_a, sem_b):
    wid = lax.axis_index("s") * NC + lax.axis_index("c")
    lanes = lax.iota(jnp.int32, 16)

    pltpu.sync_copy(pc.at[pl.ds(wid * POS_PW, POS_PW)],
                    idx_c.at[pl.ds(0, POS_PW)])
    pltpu.sync_copy(ncx.at[pl.ds(wid * NEG_PW, NEG_PW)],
                    idx_c.at[pl.ds(POS_PW, NEG_PW)])
    pltpu.sync_copy(pt.at[pl.ds(wid * POS_PW, POS_PW)],
                    idx_t.at[pl.ds(0, POS_PW)])
    pltpu.sync_copy(ntg.at[pl.ds(wid * NEG_PW, NEG_PW)],
                    idx_t.at[pl.ds(POS_PW, NEG_PW)])

    def issue(k, bc, bt, sem):
        for g in range(G):
            vc = idx_c[pl.ds(k * C + g * 16, 16)]
            vt = idx_t[pl.ds(k * C + g * 16, 16)]
            for i in range(16):
                j = g * 16 + i
                blk_c = pl.multiple_of(vc[i] & jnp.int32(~7), 8)
                blk_t = pl.multiple_of(vt[i] & jnp.int32(~7), 8)
                pltpu.async_copy(ctx.at[pl.ds(blk_c, 8), :],
                                 bc.at[pl.ds(j * 8, 8), :], sem)
                pltpu.async_copy(tgt.at[pl.ds(blk_t, 8), :],
                                 bt.at[pl.ds(j * 8, 8), :], sem)

    def drain(bc, bt, sem):
        pltpu.make_async_copy(ctx.at[pl.ds(0, C * 8), :], bc, sem).wait()
        pltpu.make_async_copy(ctx.at[pl.ds(0, C * 8), :], bt, sem).wait()

    def compute(k, bc, bt):
        for g in range(G):
            vc = idx_c[pl.ds(k * C + g * 16, 16)]
            vt = idx_t[pl.ds(k * C + g * 16, 16)]
            tot = jnp.zeros((16,), jnp.float32)
            for i in range(16):
                j = g * 16 + i
                rc = j * 8 + (vc[i] & 7)
                rt = j * 8 + (vt[i] & 7)
                acc = bc[rc, pl.ds(0, 16)] * bt[rt, pl.ds(0, 16)]
                for q in range(1, 4):
                    acc = acc + (bc[rc, pl.ds(q * 16, 16)]
                                 * bt[rt, pl.ds(q * 16, 16)])
                s = jnp.sum(acc)
                tot = jnp.where(lanes == i, s, tot)
            dots_v[pl.ds(k * C + g * 16, 16)] = tot

    issue(0, bc_a, bt_a, sem_a)

    def body(k2, carry):
        c0 = k2 * 2
        issue(c0 + 1, bc_b, bt_b, sem_b)
        drain(bc_a, bt_a, sem_a)
        compute(c0, bc_a, bt_a)

        @pl.when(k2 < NCH // 2 - 1)
        def _():
            issue(c0 + 2, bc_a, bt_a, sem_a)

        drain(bc_b, bt_b, sem_b)
        compute(c0 + 1, bc_b, bt_b)
        return carry

    lax.fori_loop(0, NCH // 2, body, 0)

    pltpu.sync_copy(dots_v.at[pl.ds(0, POS_PW)],
                    out.at[pl.ds(wid * POS_PW, POS_PW)])
    pltpu.sync_copy(dots_v.at[pl.ds(POS_PW, NEG_PW)],
                    out.at[pl.ds(B_POS + wid * NEG_PW, NEG_PW)])


def _loss_tc(dp_ref, dn_ref, out_ref):
    dp = dp_ref[...]
    dn = dn_ref[...]
    pos_loss = -jnp.mean(jnp.log(jax.nn.sigmoid(dp)))
    neg_loss = -jnp.mean(jnp.log(jax.nn.sigmoid(-dn)))
    out_ref[0, 0] = pos_loss + neg_loss


_loss_call = pl.pallas_call(
    _loss_tc,
    out_shape=jax.ShapeDtypeStruct((1, 1), jnp.float32),
    out_specs=pl.BlockSpec(memory_space=pltpu.SMEM),
)


def kernel(positive_context, positive_target, negative_context,
           negative_target, context_embeddings, target_embeddings):
    pc = positive_context.astype(jnp.int32)
    pt = positive_target.astype(jnp.int32)
    ncx = negative_context.astype(jnp.int32)
    ntg = negative_target.astype(jnp.int32)
    dots = _dots_sc(pc, pt, ncx, ntg, context_embeddings, target_embeddings)
    dp = dots[:B_POS].reshape(B_POS // 128, 128)
    dn = dots[B_POS:].reshape(B_NEG // 128, 128)
    return _loss_call(dp, dn)[0, 0]


# speedup vs baseline: 1.2618x; 1.1273x over previous
import functools

import jax
import jax.numpy as jnp
from jax import lax
from jax.experimental import pallas as pl
from jax.experimental.pallas import tpu as pltpu
from jax.experimental.pallas import tpu_sc as plsc

V = 1000000
VH = V // 2
D = 64
B_POS = 16384
B_NEG = 81920
TOTAL = B_POS + B_NEG

NC, NS = 2, 16
NW = NC * NS
POS_PW = B_POS // NW
NEG_PW = B_NEG // NW
PAIRS_PW = POS_PW + NEG_PW
C = 64
NCH = PAIRS_PW // C
G = C // 16


BT = 4096
_GRID_T = (V + BT - 1) // BT


def _pack_tc(x_ref, o_ref):
    x = x_ref[...]
    xt = x.T
    x3 = xt.reshape(BT // 2, 2, D)
    o_ref[...] = jnp.concatenate([x3[:, 0, :], x3[:, 1, :]], axis=1)


_pack_call = pl.pallas_call(
    _pack_tc,
    grid=(_GRID_T,),
    in_specs=[pl.BlockSpec((D, BT), lambda g: (0, g))],
    out_specs=pl.BlockSpec((BT // 2, 128), lambda g: (g, 0)),
    out_shape=jax.ShapeDtypeStruct((VH, 128), jnp.float32),
)


_mesh = plsc.VectorSubcoreMesh(
    core_axis_name="c", subcore_axis_name="s", num_cores=NC, num_subcores=NS)


@functools.partial(
    pl.kernel,
    out_type=jax.ShapeDtypeStruct((TOTAL,), jnp.float32),
    mesh=_mesh,
    compiler_params=pltpu.CompilerParams(needs_layout_passes=False),
    scratch_types=[
        pltpu.VMEM((PAIRS_PW,), jnp.int32),
        pltpu.VMEM((PAIRS_PW,), jnp.int32),
        pltpu.VMEM((PAIRS_PW,), jnp.int32),
        pltpu.VMEM((PAIRS_PW,), jnp.int32),
        pltpu.VMEM((C, 128), jnp.float32),
        pltpu.VMEM((C, 128), jnp.float32),
        pltpu.VMEM((C, 128), jnp.float32),
        pltpu.VMEM((C, 128), jnp.float32),
        pltpu.VMEM((PAIRS_PW,), jnp.float32),
        pltpu.SemaphoreType.DMA,
        pltpu.SemaphoreType.DMA,
    ],
)
def _dots_sc(pc, pt, ncx, ntg, ctxp, tgtp, out,
             idx_c, idx_t, idh_c, idh_t, rc_a, rt_a, rc_b, rt_b,
             dots_v, sem_a, sem_b):
    wid = lax.axis_index("s") * NC + lax.axis_index("c")
    lanes = lax.iota(jnp.int32, 16)

    pltpu.sync_copy(pc.at[pl.ds(wid * POS_PW, POS_PW)],
                    idx_c.at[pl.ds(0, POS_PW)])
    pltpu.sync_copy(ncx.at[pl.ds(wid * NEG_PW, NEG_PW)],
                    idx_c.at[pl.ds(POS_PW, NEG_PW)])
    pltpu.sync_copy(pt.at[pl.ds(wid * POS_PW, POS_PW)],
                    idx_t.at[pl.ds(0, POS_PW)])
    pltpu.sync_copy(ntg.at[pl.ds(wid * NEG_PW, NEG_PW)],
                    idx_t.at[pl.ds(POS_PW, NEG_PW)])

    def halve_body(g, carry):
        idh_c[pl.ds(g * 16, 16)] = lax.shift_right_logical(
            idx_c[pl.ds(g * 16, 16)], 1)
        idh_t[pl.ds(g * 16, 16)] = lax.shift_right_logical(
            idx_t[pl.ds(g * 16, 16)], 1)
        return carry

    lax.fori_loop(0, PAIRS_PW // 16, halve_body, 0)

    def issue(k, rc, rt, sem):
        pltpu.async_copy(ctxp.at[idh_c.at[pl.ds(k * C, C)]], rc, sem)
        pltpu.async_copy(tgtp.at[idh_t.at[pl.ds(k * C, C)]], rt, sem)

    def drain(rc, rt, sem):
        pltpu.make_async_copy(ctxp.at[pl.ds(0, C), :], rc, sem).wait()
        pltpu.make_async_copy(ctxp.at[pl.ds(0, C), :], rt, sem).wait()

    def compute(k, rc, rt):
        for g in range(G):
            vc = idx_c[pl.ds(k * C + g * 16, 16)]
            vt = idx_t[pl.ds(k * C + g * 16, 16)]
            tot = jnp.zeros((16,), jnp.float32)
            for i in range(16):
                j = g * 16 + i
                hc = (vc[i] & 1) * D
                ht = (vt[i] & 1) * D
                acc = rc[j, pl.ds(hc, 16)] * rt[j, pl.ds(ht, 16)]
                for q in range(1, 4):
                    acc = acc + (rc[j, pl.ds(hc + q * 16, 16)]
                                 * rt[j, pl.ds(ht + q * 16, 16)])
                s = jnp.sum(acc)
                tot = jnp.where(lanes == i, s, tot)
            dots_v[pl.ds(k * C + g * 16, 16)] = tot

    issue(0, rc_a, rt_a, sem_a)

    def body(k2, carry):
        c0 = k2 * 2
        issue(c0 + 1, rc_b, rt_b, sem_b)
        drain(rc_a, rt_a, sem_a)
        compute(c0, rc_a, rt_a)

        @pl.when(k2 < NCH // 2 - 1)
        def _():
            issue(c0 + 2, rc_a, rt_a, sem_a)

        drain(rc_b, rt_b, sem_b)
        compute(c0 + 1, rc_b, rt_b)
        return carry

    lax.fori_loop(0, NCH // 2, body, 0)

    pltpu.sync_copy(dots_v.at[pl.ds(0, POS_PW)],
                    out.at[pl.ds(wid * POS_PW, POS_PW)])
    pltpu.sync_copy(dots_v.at[pl.ds(POS_PW, NEG_PW)],
                    out.at[pl.ds(B_POS + wid * NEG_PW, NEG_PW)])


def _loss_tc(dp_ref, dn_ref, out_ref):
    dp = dp_ref[...]
    dn = dn_ref[...]
    pos_loss = -jnp.mean(jnp.log(jax.nn.sigmoid(dp)))
    neg_loss = -jnp.mean(jnp.log(jax.nn.sigmoid(-dn)))
    out_ref[0, 0] = pos_loss + neg_loss


_loss_call = pl.pallas_call(
    _loss_tc,
    out_shape=jax.ShapeDtypeStruct((1, 1), jnp.float32),
    out_specs=pl.BlockSpec(memory_space=pltpu.SMEM),
)


def kernel(positive_context, positive_target, negative_context,
           negative_target, context_embeddings, target_embeddings):
    pc = positive_context.astype(jnp.int32)
    pt = positive_target.astype(jnp.int32)
    ncx = negative_context.astype(jnp.int32)
    ntg = negative_target.astype(jnp.int32)
    ctxp = _pack_call(context_embeddings.T)
    tgtp = _pack_call(target_embeddings.T)
    dots = _dots_sc(pc, pt, ncx, ntg, ctxp, tgtp)
    dp = dots[:B_POS].reshape(B_POS // 128, 128)
    dn = dots[B_POS:].reshape(B_NEG // 128, 128)
    return _loss_call(dp, dn)[0, 0]
